# Initial kernel scaffold; baseline (speedup 1.0000x reference)
import jax
import jax.numpy as jnp
from jax.experimental import pallas as pl


def kernel(input_ids, word_emb, pos_emb, ln_weight, ln_bias):
    raise NotImplementedError("write your pallas kernel here")


# speedup vs baseline: 2.0321x; 2.0321x over previous
import functools

import jax
import jax.numpy as jnp
from jax import lax
from jax.experimental import pallas as pl
from jax.experimental.pallas import tpu as pltpu
from jax.experimental.pallas import tpu_sc as plsc

EPS = 1e-12
GW = 32
SBLK = 512


def _sc_gather(word_emb, ids1d, n_tokens, d):
    info = plsc.get_sparse_core_info()
    nw = info.num_cores * info.num_subcores
    per_w = n_tokens // nw
    nchunk = per_w // GW
    mesh = plsc.VectorSubcoreMesh(core_axis_name="c", subcore_axis_name="s")

    @functools.partial(
        pl.kernel,
        out_type=jax.ShapeDtypeStruct((n_tokens, d), jnp.float32),
        mesh=mesh,
        scratch_types=[
            pltpu.VMEM((per_w,), jnp.int32),
            pltpu.VMEM((2, GW, d), jnp.float32),
            pltpu.SemaphoreType.DMA((2,)),
            pltpu.SemaphoreType.DMA((2,)),
        ],
    )
    def k(table_hbm, idx_hbm, out_hbm, idx_v, buf, gsem, osem):
        wid = lax.axis_index("s") * info.num_cores + lax.axis_index("c")
        base = wid * per_w
        pltpu.sync_copy(idx_hbm.at[pl.ds(base, per_w)], idx_v)
        handles_o = [None] * nchunk
        for i in range(nchunk):
            b = i % 2
            if i >= 2:
                handles_o[i - 2].wait()
            g = pltpu.async_copy(
                table_hbm.at[idx_v.at[pl.ds(i * GW, GW)]], buf.at[b], gsem.at[b]
            )
            g.wait()
            handles_o[i] = pltpu.async_copy(
                buf.at[b], out_hbm.at[pl.ds(base + i * GW, GW)], osem.at[b]
            )
        for i in range(max(nchunk - 2, 0), nchunk):
            handles_o[i].wait()

    return k(word_emb, ids1d)


def _ln_transpose_body(g_ref, p_ref, w_ref, b_ref, o_ref):
    x = g_ref[...] + p_ref[...]
    u = jnp.mean(x, axis=1, keepdims=True)
    dlt = x - u
    v = jnp.mean(dlt * dlt, axis=1, keepdims=True)
    y = dlt * lax.rsqrt(v + EPS)
    y = y * w_ref[...] + b_ref[...]
    o_ref[0] = y.T


def _tc_ln_transpose(gathered, pos_emb, ln_weight, ln_bias, bsz, s, d):
    ns = s // SBLK
    return pl.pallas_call(
        _ln_transpose_body,
        grid=(ns, bsz),
        in_specs=[
            pl.BlockSpec((SBLK, d), lambda i, b: (b * ns + i, 0)),
            pl.BlockSpec((SBLK, d), lambda i, b: (i, 0)),
            pl.BlockSpec((1, d), lambda i, b: (0, 0)),
            pl.BlockSpec((1, d), lambda i, b: (0, 0)),
        ],
        out_specs=pl.BlockSpec((1, d, SBLK), lambda i, b: (b, 0, i)),
        out_shape=jax.ShapeDtypeStruct((bsz, d, s), jnp.float32),
        compiler_params=pltpu.CompilerParams(
            dimension_semantics=("parallel", "parallel"),
        ),
    )(gathered, pos_emb, ln_weight.reshape(1, d), ln_bias.reshape(1, d))


def kernel(input_ids, word_emb, pos_emb, ln_weight, ln_bias):
    bsz, s = input_ids.shape
    _, d = word_emb.shape
    ids1d = input_ids.astype(jnp.int32).reshape(bsz * s)
    gathered = _sc_gather(word_emb, ids1d, bsz * s, d)
    return _tc_ln_transpose(gathered, pos_emb, ln_weight, ln_bias, bsz, s, d)


# speedup vs baseline: 2.1436x; 1.0549x over previous
import functools

import jax
import jax.numpy as jnp
from jax import lax
from jax.experimental import pallas as pl
from jax.experimental.pallas import tpu as pltpu
from jax.experimental.pallas import tpu_sc as plsc

EPS = 1e-12
GW = 32
SBLK = 1024


def _sc_gather(word_emb, ids1d, n_tokens, d):
    info = plsc.get_sparse_core_info()
    nw = info.num_cores * info.num_subcores
    per_w = n_tokens // nw
    nchunk = per_w // GW
    mesh = plsc.VectorSubcoreMesh(core_axis_name="c", subcore_axis_name="s")

    @functools.partial(
        pl.kernel,
        out_type=jax.ShapeDtypeStruct((n_tokens, d), jnp.float32),
        mesh=mesh,
        scratch_types=[
            pltpu.VMEM((per_w,), jnp.int32),
            pltpu.VMEM((2, GW, d), jnp.float32),
            pltpu.SemaphoreType.DMA((2,)),
            pltpu.SemaphoreType.DMA((2,)),
        ],
    )
    def k(table_hbm, idx_hbm, out_hbm, idx_v, buf, gsem, osem):
        wid = lax.axis_index("s") * info.num_cores + lax.axis_index("c")
        base = wid * per_w
        pltpu.sync_copy(idx_hbm.at[pl.ds(base, per_w)], idx_v)
        handles_o = [None] * nchunk
        for i in range(nchunk):
            b = i % 2
            if i >= 2:
                handles_o[i - 2].wait()
            g = pltpu.async_copy(
                table_hbm.at[idx_v.at[pl.ds(i * GW, GW)]], buf.at[b], gsem.at[b]
            )
            g.wait()
            handles_o[i] = pltpu.async_copy(
                buf.at[b], out_hbm.at[pl.ds(base + i * GW, GW)], osem.at[b]
            )
        for i in range(max(nchunk - 2, 0), nchunk):
            handles_o[i].wait()

    return k(word_emb, ids1d)


def _ln_transpose_body(g_ref, p_ref, w_ref, b_ref, o_ref):
    x = g_ref[...] + p_ref[...]
    u = jnp.mean(x, axis=1, keepdims=True)
    dlt = x - u
    v = jnp.mean(dlt * dlt, axis=1, keepdims=True)
    y = dlt * lax.rsqrt(v + EPS)
    y = y * w_ref[...] + b_ref[...]
    o_ref[0] = y.T


def _tc_ln_transpose(gathered, pos_emb, ln_weight, ln_bias, bsz, s, d):
    ns = s // SBLK
    return pl.pallas_call(
        _ln_transpose_body,
        grid=(ns, bsz),
        in_specs=[
            pl.BlockSpec((SBLK, d), lambda i, b: (b * ns + i, 0)),
            pl.BlockSpec((SBLK, d), lambda i, b: (i, 0)),
            pl.BlockSpec((1, d), lambda i, b: (0, 0)),
            pl.BlockSpec((1, d), lambda i, b: (0, 0)),
        ],
        out_specs=pl.BlockSpec((1, d, SBLK), lambda i, b: (b, 0, i)),
        out_shape=jax.ShapeDtypeStruct((bsz, d, s), jnp.float32),
        compiler_params=pltpu.CompilerParams(
            dimension_semantics=("parallel", "parallel"),
        ),
    )(gathered, pos_emb, ln_weight.reshape(1, d), ln_bias.reshape(1, d))


def kernel(input_ids, word_emb, pos_emb, ln_weight, ln_bias):
    bsz, s = input_ids.shape
    _, d = word_emb.shape
    ids1d = input_ids.astype(jnp.int32).reshape(bsz * s)
    gathered = _sc_gather(word_emb, ids1d, bsz * s, d)
    return _tc_ln_transpose(gathered, pos_emb, ln_weight, ln_bias, bsz, s, d)


# speedup vs baseline: 2.1751x; 1.0147x over previous
import functools

import jax
import jax.numpy as jnp
from jax import lax
from jax.experimental import pallas as pl
from jax.experimental.pallas import tpu as pltpu
from jax.experimental.pallas import tpu_sc as plsc

EPS = 1e-12
GW = 32
SBLK = 2048


def _sc_gather(word_emb, ids1d, n_tokens, d):
    info = plsc.get_sparse_core_info()
    nw = info.num_cores * info.num_subcores
    per_w = n_tokens // nw
    nchunk = per_w // GW
    mesh = plsc.VectorSubcoreMesh(core_axis_name="c", subcore_axis_name="s")

    @functools.partial(
        pl.kernel,
        out_type=jax.ShapeDtypeStruct((n_tokens, d), jnp.float32),
        mesh=mesh,
        scratch_types=[
            pltpu.VMEM((per_w,), jnp.int32),
            pltpu.VMEM((2, GW, d), jnp.float32),
            pltpu.SemaphoreType.DMA((2,)),
            pltpu.SemaphoreType.DMA((2,)),
        ],
    )
    def k(table_hbm, idx_hbm, out_hbm, idx_v, buf, gsem, osem):
        wid = lax.axis_index("s") * info.num_cores + lax.axis_index("c")
        base = wid * per_w
        pltpu.sync_copy(idx_hbm.at[pl.ds(base, per_w)], idx_v)
        handles_o = [None] * nchunk
        for i in range(nchunk):
            b = i % 2
            if i >= 2:
                handles_o[i - 2].wait()
            g = pltpu.async_copy(
                table_hbm.at[idx_v.at[pl.ds(i * GW, GW)]], buf.at[b], gsem.at[b]
            )
            g.wait()
            handles_o[i] = pltpu.async_copy(
                buf.at[b], out_hbm.at[pl.ds(base + i * GW, GW)], osem.at[b]
            )
        for i in range(max(nchunk - 2, 0), nchunk):
            handles_o[i].wait()

    return k(word_emb, ids1d)


def _ln_transpose_body(g_ref, p_ref, w_ref, b_ref, o_ref):
    x = g_ref[...] + p_ref[...]
    u = jnp.mean(x, axis=1, keepdims=True)
    dlt = x - u
    v = jnp.mean(dlt * dlt, axis=1, keepdims=True)
    y = dlt * lax.rsqrt(v + EPS)
    y = y * w_ref[...] + b_ref[...]
    o_ref[0] = y.T


def _tc_ln_transpose(gathered, pos_emb, ln_weight, ln_bias, bsz, s, d):
    ns = s // SBLK
    return pl.pallas_call(
        _ln_transpose_body,
        grid=(ns, bsz),
        in_specs=[
            pl.BlockSpec((SBLK, d), lambda i, b: (b * ns + i, 0)),
            pl.BlockSpec((SBLK, d), lambda i, b: (i, 0)),
            pl.BlockSpec((1, d), lambda i, b: (0, 0)),
            pl.BlockSpec((1, d), lambda i, b: (0, 0)),
        ],
        out_specs=pl.BlockSpec((1, d, SBLK), lambda i, b: (b, 0, i)),
        out_shape=jax.ShapeDtypeStruct((bsz, d, s), jnp.float32),
        compiler_params=pltpu.CompilerParams(
            dimension_semantics=("parallel", "parallel"),
        ),
    )(gathered, pos_emb, ln_weight.reshape(1, d), ln_bias.reshape(1, d))


def kernel(input_ids, word_emb, pos_emb, ln_weight, ln_bias):
    bsz, s = input_ids.shape
    _, d = word_emb.shape
    ids1d = input_ids.astype(jnp.int32).reshape(bsz * s)
    gathered = _sc_gather(word_emb, ids1d, bsz * s, d)
    return _tc_ln_transpose(gathered, pos_emb, ln_weight, ln_bias, bsz, s, d)
